# baseline (speedup 1.0000x reference)
import functools

import jax
import jax.numpy as jnp
import numpy as np
from jax.experimental import pallas as pl
from jax.experimental.pallas import tpu as pltpu

_MAX_THRESHOLD = 0.95
_LOG2E = 1.4426950408889634
_SUB = 8


def _or_layer_kernel(tau_ref, x_ref, w_ref, o_ref, *, tau_floor, o_blk):
    ta = tau_ref[0, 0]
    tau = tau_floor + jnp.where(ta >= 0, ta, 0.05 * ta)
    ts = tau * _LOG2E
    inv_ts = 1.0 / ts
    x = x_ref[...]
    w = w_ref[...]
    aw = jnp.where(w < 0.0, 0.1 * w,
                   jnp.where(w > 1.0, 1.0 + 0.1 * (w - 1.0), w))
    awt = aw * ts
    C = jnp.max(jnp.abs(awt)) * jnp.max(jnp.abs(x))
    for j in range(o_blk // _SUB):
        awj = awt[j * _SUB:(j + 1) * _SUB, :]
        t = awj[:, None, :] * x[None, :, :]
        u = t - C
        p = jnp.exp2(u)
        den = jnp.sum(p, axis=-1, keepdims=True)
        num = jnp.sum(p * u, axis=-1, keepdims=True)
        o_ref[j * _SUB:(j + 1) * _SUB, :] = (
            1.0 - inv_ts * (C + (num / den)[:, :, 0]))


def _or_layer_t(h, W, tau_adder, o_blk):
    B, IN = h.shape
    OUT = W.shape[0]
    tau_floor = float(np.log(IN - 1) + np.log(_MAX_THRESHOLD)
                      - np.log(1.0 - _MAX_THRESHOLD))
    tau2d = tau_adder.reshape(1, 1)
    return pl.pallas_call(
        functools.partial(_or_layer_kernel, tau_floor=tau_floor, o_blk=o_blk),
        out_shape=jax.ShapeDtypeStruct((OUT, B), jnp.float32),
        grid=(OUT // o_blk,),
        in_specs=[
            pl.BlockSpec(memory_space=pltpu.SMEM),
            pl.BlockSpec((B, IN), lambda o: (0, 0)),
            pl.BlockSpec((o_blk, IN), lambda o: (o, 0)),
        ],
        out_specs=pl.BlockSpec((o_blk, B), lambda o: (o, 0)),
        compiler_params=pltpu.CompilerParams(
            dimension_semantics=("arbitrary",),
            vmem_limit_bytes=48 * 1024 * 1024,
        ),
        name="or_gate_layer",
    )(tau2d, h, W)


def kernel(x, W0, W1, W2, W3, tau0, tau1, tau2, tau3):
    h = jnp.concatenate([x, 1.0 - x], axis=-1)
    for W, t, blk in ((W0, tau0, 32), (W1, tau1, 32),
                      (W2, tau2, 32), (W3, tau3, 32)):
        h = _or_layer_t(h, W, t, blk).T
    return h


# speedup vs baseline: 1.0028x; 1.0028x over previous
import functools

import jax
import jax.numpy as jnp
import numpy as np
from jax.experimental import pallas as pl
from jax.experimental.pallas import tpu as pltpu

_MAX_THRESHOLD = 0.95
_LOG2E = 1.4426950408889634
_SUB = 16


def _or_layer_kernel(tau_ref, x_ref, w_ref, o_ref, *, tau_floor, o_blk):
    ta = tau_ref[0, 0]
    tau = tau_floor + jnp.where(ta >= 0, ta, 0.05 * ta)
    ts = tau * _LOG2E
    inv_ts = 1.0 / ts
    x = x_ref[...]
    w = w_ref[...]
    aw = jnp.where(w < 0.0, 0.1 * w,
                   jnp.where(w > 1.0, 1.0 + 0.1 * (w - 1.0), w))
    awt = aw * ts
    C = jnp.max(jnp.abs(awt)) * jnp.max(jnp.abs(x))
    for j in range(o_blk // _SUB):
        awj = awt[j * _SUB:(j + 1) * _SUB, :]
        t = awj[:, None, :] * x[None, :, :]
        u = t - C
        p = jnp.exp2(u)
        den = jnp.sum(p, axis=-1, keepdims=True)
        num = jnp.sum(p * u, axis=-1, keepdims=True)
        o_ref[j * _SUB:(j + 1) * _SUB, :] = (
            1.0 - inv_ts * (C + (num / den)[:, :, 0]))


def _or_layer_t(h, W, tau_adder, o_blk):
    B, IN = h.shape
    OUT = W.shape[0]
    tau_floor = float(np.log(IN - 1) + np.log(_MAX_THRESHOLD)
                      - np.log(1.0 - _MAX_THRESHOLD))
    tau2d = tau_adder.reshape(1, 1)
    return pl.pallas_call(
        functools.partial(_or_layer_kernel, tau_floor=tau_floor, o_blk=o_blk),
        out_shape=jax.ShapeDtypeStruct((OUT, B), jnp.float32),
        grid=(OUT // o_blk,),
        in_specs=[
            pl.BlockSpec(memory_space=pltpu.SMEM),
            pl.BlockSpec((B, IN), lambda o: (0, 0)),
            pl.BlockSpec((o_blk, IN), lambda o: (o, 0)),
        ],
        out_specs=pl.BlockSpec((o_blk, B), lambda o: (o, 0)),
        compiler_params=pltpu.CompilerParams(
            dimension_semantics=("arbitrary",),
            vmem_limit_bytes=48 * 1024 * 1024,
        ),
        name="or_gate_layer",
    )(tau2d, h, W)


def kernel(x, W0, W1, W2, W3, tau0, tau1, tau2, tau3):
    h = jnp.concatenate([x, 1.0 - x], axis=-1)
    for W, t, blk in ((W0, tau0, 64), (W1, tau1, 64),
                      (W2, tau2, 32), (W3, tau3, 32)):
        h = _or_layer_t(h, W, t, blk).T
    return h


# speedup vs baseline: 1.0092x; 1.0064x over previous
import functools

import jax
import jax.numpy as jnp
import numpy as np
from jax.experimental import pallas as pl
from jax.experimental.pallas import tpu as pltpu

_MAX_THRESHOLD = 0.95
_LOG2E = 1.4426950408889634
_SUB = 8


def _or_layer_kernel(tau_ref, x_ref, w_ref, o_ref, *, tau_floor, o_blk):
    ta = tau_ref[0, 0]
    tau = tau_floor + jnp.where(ta >= 0, ta, 0.05 * ta)
    ts = tau * _LOG2E
    inv_ts = 1.0 / ts
    x = x_ref[...]
    w = w_ref[...]
    aw = jnp.where(w < 0.0, 0.1 * w,
                   jnp.where(w > 1.0, 1.0 + 0.1 * (w - 1.0), w))
    awt = aw * ts
    C = jnp.max(jnp.abs(awt)) * jnp.max(jnp.abs(x))
    for j in range(o_blk // _SUB):
        awj = awt[j * _SUB:(j + 1) * _SUB, :]
        t = awj[:, None, :] * x[None, :, :]
        u = t - C
        p = jnp.exp2(u)
        den = jnp.sum(p, axis=-1, keepdims=True)[:, :, 0]
        num = jnp.sum(p * u, axis=-1, keepdims=True)[:, :, 0]
        o_ref[j * _SUB:(j + 1) * _SUB, :] = (
            1.0 - inv_ts * (C + num / den))


def _or_layer_t(h, W, tau_adder, o_blk):
    B, IN = h.shape
    OUT = W.shape[0]
    tau_floor = float(np.log(IN - 1) + np.log(_MAX_THRESHOLD)
                      - np.log(1.0 - _MAX_THRESHOLD))
    tau2d = tau_adder.reshape(1, 1)
    return pl.pallas_call(
        functools.partial(_or_layer_kernel, tau_floor=tau_floor, o_blk=o_blk),
        out_shape=jax.ShapeDtypeStruct((OUT, B), jnp.float32),
        grid=(OUT // o_blk,),
        in_specs=[
            pl.BlockSpec(memory_space=pltpu.SMEM),
            pl.BlockSpec((B, IN), lambda o: (0, 0)),
            pl.BlockSpec((o_blk, IN), lambda o: (o, 0)),
        ],
        out_specs=pl.BlockSpec((o_blk, B), lambda o: (o, 0)),
        compiler_params=pltpu.CompilerParams(
            dimension_semantics=("arbitrary",),
            vmem_limit_bytes=48 * 1024 * 1024,
        ),
        name="or_gate_layer",
    )(tau2d, h, W)


def kernel(x, W0, W1, W2, W3, tau0, tau1, tau2, tau3):
    h = jnp.concatenate([x, 1.0 - x], axis=-1)
    for W, t, blk in ((W0, tau0, 64), (W1, tau1, 64),
                      (W2, tau2, 32), (W3, tau3, 32)):
        h = _or_layer_t(h, W, t, blk).T
    return h
